# Initial kernel scaffold; baseline (speedup 1.0000x reference)
import jax
import jax.numpy as jnp
from jax.experimental import pallas as pl


def kernel(x, W_in, b_in, g_in, be_in, W1, b1, W2, b2, W_out, b_out, g_out, be_out):
    raise NotImplementedError("write your pallas kernel here")


# speedup vs baseline: 7.1943x; 7.1943x over previous
import functools

import jax
import jax.numpy as jnp
from jax import lax
from jax.experimental import pallas as pl
from jax.experimental.pallas import tpu as pltpu
from jax.experimental.pallas import tpu_sc as plsc

C = 192
CP = 256
K = 9
N = 1024
B = 8
RB = 256
NB = N // RB
NTOT = B * N
F32 = jnp.float32
HI = lax.Precision.HIGHEST


def _stats_accum(s_ref, h, is_first):
    s = jnp.sum(h, axis=0, keepdims=True)
    ss = jnp.sum(h * h, axis=0, keepdims=True)
    contrib = jnp.concatenate([s, ss, jnp.zeros((6, C), F32)], axis=0)

    @pl.when(is_first)
    def _():
        s_ref[...] = contrib

    @pl.when(jnp.logical_not(is_first))
    def _():
        s_ref[...] = s_ref[...] + contrib


def _bn_apply(s_ref, h, g_ref, be_ref):
    mean = s_ref[0:1, :] * (1.0 / NTOT)
    var = s_ref[1:2, :] * (1.0 / NTOT) - mean * mean
    inv = 1.0 / jnp.sqrt(var + 1e-5)
    return (h - mean) * inv * g_ref[...] + be_ref[...]


def _k1(x_ref, w_ref, b_ref, h_ref, s_ref):
    b = pl.program_id(0)
    j = pl.program_id(1)
    h = jnp.dot(x_ref[0], w_ref[...],
                preferred_element_type=F32) + b_ref[...]
    h_ref[0] = h
    _stats_accum(s_ref, h, jnp.logical_and(b == 0, j == 0))


def _call1(x2, W_inT, b_in2):
    return pl.pallas_call(
        _k1,
        grid=(B, NB),
        in_specs=[
            pl.BlockSpec((1, RB, C), lambda b, j: (b, j, 0)),
            pl.BlockSpec((C, C), lambda b, j: (0, 0)),
            pl.BlockSpec((1, C), lambda b, j: (0, 0)),
        ],
        out_specs=[
            pl.BlockSpec((1, RB, C), lambda b, j: (b, j, 0)),
            pl.BlockSpec((8, C), lambda b, j: (0, 0)),
        ],
        out_shape=[
            jax.ShapeDtypeStruct((B, N, C), F32),
            jax.ShapeDtypeStruct((8, C), F32),
        ],
    )(x2, W_inT, b_in2)


def _k2(h_ref, s_ref, g_ref, be_ref, wd_ref, wb_ref, b1_ref,
        f_ref, p_ref, q_ref):
    xh = _bn_apply(s_ref, h_ref[0], g_ref, be_ref)
    f = 0.5 * xh * (1.0 + lax.erf(xh * 0.7071067811865476))
    f_ref[0] = f
    p_ref[0] = jnp.dot(f, wd_ref[...], preferred_element_type=F32,
                       precision=HI) + b1_ref[...]
    q_ref[0] = jnp.dot(f, wb_ref[...], preferred_element_type=F32,
                       precision=HI)


def _call2(H1, S1, g2, be2, W1dT, W1bT, b12):
    return pl.pallas_call(
        _k2,
        grid=(B, NB),
        in_specs=[
            pl.BlockSpec((1, RB, C), lambda b, j: (b, j, 0)),
            pl.BlockSpec((8, C), lambda b, j: (0, 0)),
            pl.BlockSpec((1, C), lambda b, j: (0, 0)),
            pl.BlockSpec((1, C), lambda b, j: (0, 0)),
            pl.BlockSpec((C, CP), lambda b, j: (0, 0)),
            pl.BlockSpec((C, CP), lambda b, j: (0, 0)),
            pl.BlockSpec((1, CP), lambda b, j: (0, 0)),
        ],
        out_specs=[
            pl.BlockSpec((1, RB, C), lambda b, j: (b, j, 0)),
            pl.BlockSpec((1, RB, CP), lambda b, j: (b, j, 0)),
            pl.BlockSpec((1, RB, CP), lambda b, j: (b, j, 0)),
        ],
        out_shape=[
            jax.ShapeDtypeStruct((B, N, C), F32),
            jax.ShapeDtypeStruct((B, N, CP), F32),
            jax.ShapeDtypeStruct((B, N, CP), F32),
        ],
    )(H1, S1, g2, be2, W1dT, W1bT, b12)


def _k3(fall_ref, frow_ref, idx_ref):
    b = pl.program_id(0)
    fa = fall_ref[0]
    fr = frow_ref[0]
    S = lax.dot_general(fr, fa, (((1,), (1,)), ((), ())),
                        preferred_element_type=F32)
    sqr = jnp.sum(fr * fr, axis=1, keepdims=True)
    sqa = jnp.sum(fa * fa, axis=1).reshape(1, N)
    d = jnp.sqrt(jnp.maximum(sqr + sqa - 2.0 * S, 0.0))
    iota = lax.broadcasted_iota(jnp.int32, (RB, N), 1)
    base = b * N
    rows = []
    for _ in range(K):
        m = jnp.min(d, axis=1, keepdims=True)
        ik = jnp.min(jnp.where(d == m, iota, jnp.int32(2**30)), axis=1)
        rows.append(ik + base)
        d = jnp.where(iota == ik[:, None], jnp.float32(3e38), d)
    idx_ref[0] = jnp.stack(rows, axis=0)


def _call3(feats):
    return pl.pallas_call(
        _k3,
        grid=(B, NB),
        in_specs=[
            pl.BlockSpec((1, N, C), lambda b, j: (b, 0, 0)),
            pl.BlockSpec((1, RB, C), lambda b, j: (b, j, 0)),
        ],
        out_specs=[pl.BlockSpec((1, K, RB), lambda b, j: (b, 0, j))],
        out_shape=[jax.ShapeDtypeStruct((B, K, N), jnp.int32)],
    )(feats, feats)[0]


NROWS = B * K * N
NW = 32
RPW = NROWS // NW
CH = 256
NCH = RPW // CH


def _sc_gather_body(q_hbm, idx_hbm, out_hbm, idx_v, rows_v, sem):
    wid = lax.axis_index("s") * 2 + lax.axis_index("c")
    base = wid * RPW
    pltpu.sync_copy(idx_hbm.at[pl.ds(base, RPW)], idx_v)
    for ch in range(NCH):
        pltpu.async_copy(q_hbm.at[idx_v.at[pl.ds(ch * CH, CH)]],
                         rows_v, sem).wait()
        pltpu.sync_copy(rows_v, out_hbm.at[pl.ds(base + ch * CH, CH)])


@functools.lru_cache(maxsize=1)
def _sc_gather_fn():
    return pl.kernel(
        _sc_gather_body,
        out_type=jax.ShapeDtypeStruct((NROWS, CP), F32),
        mesh=plsc.VectorSubcoreMesh(core_axis_name="c", subcore_axis_name="s"),
        scratch_types=[
            pltpu.VMEM((RPW,), jnp.int32),
            pltpu.VMEM((CH, CP), F32),
            pltpu.SemaphoreType.DMA,
        ],
    )


def _sc_gather(q, idx):
    return _sc_gather_fn()(q, idx)


def _k5(g_ref, p_ref, w2_ref, wo_ref, bo_ref, o_ref, s_ref):
    b = pl.program_id(0)
    j = pl.program_id(1)
    p = p_ref[0]
    acc = None
    for k in range(K):
        e = g_ref[0, k] + p
        e = jnp.where(e >= 0.0, e, 0.2 * e)
        y = jnp.dot(e, w2_ref[...], preferred_element_type=F32, precision=HI)
        acc = y if acc is None else jnp.maximum(acc, y)
    o = jnp.dot(acc, wo_ref[...], preferred_element_type=F32,
                precision=HI) + bo_ref[...]
    o_ref[0] = o
    _stats_accum(s_ref, o, jnp.logical_and(b == 0, j == 0))


def _call5(G, P, W2T, WoutT, bo2):
    return pl.pallas_call(
        _k5,
        grid=(B, NB),
        in_specs=[
            pl.BlockSpec((1, K, RB, CP), lambda b, j: (b, 0, j, 0)),
            pl.BlockSpec((1, RB, CP), lambda b, j: (b, j, 0)),
            pl.BlockSpec((CP, C), lambda b, j: (0, 0)),
            pl.BlockSpec((C, C), lambda b, j: (0, 0)),
            pl.BlockSpec((1, C), lambda b, j: (0, 0)),
        ],
        out_specs=[
            pl.BlockSpec((1, RB, C), lambda b, j: (b, j, 0)),
            pl.BlockSpec((8, C), lambda b, j: (0, 0)),
        ],
        out_shape=[
            jax.ShapeDtypeStruct((B, N, C), F32),
            jax.ShapeDtypeStruct((8, C), F32),
        ],
    )(G, P, W2T, WoutT, bo2)


def _k6(o_ref, s_ref, g_ref, be_ref, x_ref, f_ref):
    o = _bn_apply(s_ref, o_ref[0], g_ref, be_ref)
    f_ref[0] = jnp.maximum(o + x_ref[0], 0.0)


def _call6(O1, S2, g2, be2, x2):
    return pl.pallas_call(
        _k6,
        grid=(B, NB),
        in_specs=[
            pl.BlockSpec((1, RB, C), lambda b, j: (b, j, 0)),
            pl.BlockSpec((8, C), lambda b, j: (0, 0)),
            pl.BlockSpec((1, C), lambda b, j: (0, 0)),
            pl.BlockSpec((1, C), lambda b, j: (0, 0)),
            pl.BlockSpec((1, RB, C), lambda b, j: (b, j, 0)),
        ],
        out_specs=[pl.BlockSpec((1, RB, C), lambda b, j: (b, j, 0))],
        out_shape=[jax.ShapeDtypeStruct((B, N, C), F32)],
    )(O1, S2, g2, be2, x2)[0]


def kernel(x, W_in, b_in, g_in, be_in, W1, b1, W2, b2, W_out, b_out,
           g_out, be_out):
    Bx, Cc, Hh, Ww = x.shape
    x2 = x.reshape(Bx, Cc, Hh * Ww).transpose(0, 2, 1)
    W_inT = W_in.T
    pad = lambda w: jnp.pad(w, ((0, 0), (0, CP - Cc)))
    W1dT = pad((W1[:, :Cc] - W1[:, Cc:]).T)
    W1bT = pad(W1[:, Cc:].T)
    W2T = jnp.pad(W2.T, ((0, CP - Cc), (0, 0)))
    WoutT = W_out.T
    row = lambda v: v.reshape(1, Cc)

    H1, S1 = _call1(x2, W_inT, row(b_in))
    feats, P, Q = _call2(H1, S1, row(g_in), row(be_in), W1dT, W1bT,
                         jnp.pad(b1, (0, CP - Cc)).reshape(1, CP))
    idxT = _call3(feats)
    Gf = _sc_gather(Q.reshape(Bx * Hh * Ww, CP), idxT.reshape(NROWS))
    G = Gf.reshape(Bx, K, Hh * Ww, CP)
    bo_eff = (b_out + jnp.dot(b2, WoutT)).reshape(1, Cc)
    O1, S2 = _call5(G, P, W2T, WoutT, bo_eff)
    F = _call6(O1, S2, row(g_out), row(be_out), x2)
    return F.transpose(0, 2, 1).reshape(Bx, Cc, Hh, Ww)


# speedup vs baseline: 8.6024x; 1.1957x over previous
import functools

import jax
import jax.numpy as jnp
from jax import lax
from jax.experimental import pallas as pl
from jax.experimental.pallas import tpu as pltpu
from jax.experimental.pallas import tpu_sc as plsc

C = 192
CP = 256
K = 9
N = 1024
B = 8
RB = 256
NB = N // RB
NTOT = B * N
F32 = jnp.float32
HI = lax.Precision.HIGHEST


def _stats_accum(s_ref, h, is_first):
    s = jnp.sum(h, axis=0, keepdims=True)
    ss = jnp.sum(h * h, axis=0, keepdims=True)
    contrib = jnp.concatenate([s, ss, jnp.zeros((6, C), F32)], axis=0)

    @pl.when(is_first)
    def _():
        s_ref[...] = contrib

    @pl.when(jnp.logical_not(is_first))
    def _():
        s_ref[...] = s_ref[...] + contrib


def _bn_apply(s_ref, h, g_ref, be_ref):
    mean = s_ref[0:1, :] * (1.0 / NTOT)
    var = s_ref[1:2, :] * (1.0 / NTOT) - mean * mean
    inv = 1.0 / jnp.sqrt(var + 1e-5)
    return (h - mean) * inv * g_ref[...] + be_ref[...]


def _k1(x_ref, w_ref, b_ref, h_ref, s_ref):
    b = pl.program_id(0)
    j = pl.program_id(1)
    h = jnp.dot(x_ref[0], w_ref[...],
                preferred_element_type=F32) + b_ref[...]
    h_ref[0] = h
    _stats_accum(s_ref, h, jnp.logical_and(b == 0, j == 0))


def _call1(x2, W_inT, b_in2):
    return pl.pallas_call(
        _k1,
        grid=(B, NB),
        in_specs=[
            pl.BlockSpec((1, RB, C), lambda b, j: (b, j, 0)),
            pl.BlockSpec((C, C), lambda b, j: (0, 0)),
            pl.BlockSpec((1, C), lambda b, j: (0, 0)),
        ],
        out_specs=[
            pl.BlockSpec((1, RB, C), lambda b, j: (b, j, 0)),
            pl.BlockSpec((8, C), lambda b, j: (0, 0)),
        ],
        out_shape=[
            jax.ShapeDtypeStruct((B, N, C), F32),
            jax.ShapeDtypeStruct((8, C), F32),
        ],
    )(x2, W_inT, b_in2)


def _k2(h_ref, s_ref, g_ref, be_ref, wd_ref, wb_ref, b1_ref,
        f_ref, p_ref, q_ref):
    xh = _bn_apply(s_ref, h_ref[0], g_ref, be_ref)
    f = 0.5 * xh * (1.0 + lax.erf(xh * 0.7071067811865476))
    f_ref[0] = f
    p_ref[0] = jnp.dot(f, wd_ref[...],
                       preferred_element_type=F32) + b1_ref[...]
    q_ref[0] = jnp.dot(f, wb_ref[...],
                       preferred_element_type=F32)


def _call2(H1, S1, g2, be2, W1dT, W1bT, b12):
    return pl.pallas_call(
        _k2,
        grid=(B, NB),
        in_specs=[
            pl.BlockSpec((1, RB, C), lambda b, j: (b, j, 0)),
            pl.BlockSpec((8, C), lambda b, j: (0, 0)),
            pl.BlockSpec((1, C), lambda b, j: (0, 0)),
            pl.BlockSpec((1, C), lambda b, j: (0, 0)),
            pl.BlockSpec((C, CP), lambda b, j: (0, 0)),
            pl.BlockSpec((C, CP), lambda b, j: (0, 0)),
            pl.BlockSpec((1, CP), lambda b, j: (0, 0)),
        ],
        out_specs=[
            pl.BlockSpec((1, RB, C), lambda b, j: (b, j, 0)),
            pl.BlockSpec((1, RB, CP), lambda b, j: (b, j, 0)),
            pl.BlockSpec((1, RB, CP), lambda b, j: (b, j, 0)),
        ],
        out_shape=[
            jax.ShapeDtypeStruct((B, N, C), F32),
            jax.ShapeDtypeStruct((B, N, CP), F32),
            jax.ShapeDtypeStruct((B, N, CP), F32),
        ],
    )(H1, S1, g2, be2, W1dT, W1bT, b12)


def _k3(fall_ref, frow_ref, idx_ref):
    b = pl.program_id(0)
    fa = fall_ref[0]
    fr = frow_ref[0]
    S = lax.dot_general(fr, fa, (((1,), (1,)), ((), ())),
                        preferred_element_type=F32)
    sqr = jnp.sum(fr * fr, axis=1, keepdims=True)
    sqa = jnp.sum(fa * fa, axis=1).reshape(1, N)
    d = jnp.maximum(sqr + sqa - 2.0 * S, 0.0)
    iota = lax.broadcasted_iota(jnp.int32, (RB, N), 1)
    base = b * N
    rows = []
    for _ in range(K):
        m = jnp.min(d, axis=1, keepdims=True)
        ik = jnp.min(jnp.where(d == m, iota, jnp.int32(2**30)), axis=1)
        rows.append(ik + base)
        d = jnp.where(iota == ik[:, None], jnp.float32(3e38), d)
    idx_ref[0] = jnp.stack(rows, axis=0)


def _call3(feats):
    return pl.pallas_call(
        _k3,
        grid=(B, NB),
        in_specs=[
            pl.BlockSpec((1, N, C), lambda b, j: (b, 0, 0)),
            pl.BlockSpec((1, RB, C), lambda b, j: (b, j, 0)),
        ],
        out_specs=[pl.BlockSpec((1, K, RB), lambda b, j: (b, 0, j))],
        out_shape=[jax.ShapeDtypeStruct((B, K, N), jnp.int32)],
    )(feats, feats)[0]


NROWS = B * K * N
NW = 32
RPW = NROWS // NW
CH = 256
NCH = RPW // CH


def _sc_gather_body(q_hbm, idx_hbm, out_hbm, idx_v, rows_v, sem):
    wid = lax.axis_index("s") * 2 + lax.axis_index("c")
    base = wid * RPW
    pltpu.sync_copy(idx_hbm.at[pl.ds(base, RPW)], idx_v)
    for ch in range(NCH):
        pltpu.async_copy(q_hbm.at[idx_v.at[pl.ds(ch * CH, CH)]],
                         rows_v, sem).wait()
        pltpu.sync_copy(rows_v, out_hbm.at[pl.ds(base + ch * CH, CH)])


@functools.lru_cache(maxsize=1)
def _sc_gather_fn():
    return pl.kernel(
        _sc_gather_body,
        out_type=jax.ShapeDtypeStruct((NROWS, CP), F32),
        mesh=plsc.VectorSubcoreMesh(core_axis_name="c", subcore_axis_name="s"),
        scratch_types=[
            pltpu.VMEM((RPW,), jnp.int32),
            pltpu.VMEM((CH, CP), F32),
            pltpu.SemaphoreType.DMA,
        ],
    )


def _sc_gather(q, idx):
    return _sc_gather_fn()(q, idx)


def _k5(g_ref, p_ref, w2_ref, wo_ref, bo_ref, o_ref, s_ref):
    b = pl.program_id(0)
    j = pl.program_id(1)
    p = p_ref[0]
    acc = None
    for k in range(K):
        e = g_ref[0, k] + p
        e = jnp.where(e >= 0.0, e, 0.2 * e)
        y = jnp.dot(e, w2_ref[...], preferred_element_type=F32)
        acc = y if acc is None else jnp.maximum(acc, y)
    o = jnp.dot(acc, wo_ref[...],
                preferred_element_type=F32) + bo_ref[...]
    o_ref[0] = o
    _stats_accum(s_ref, o, jnp.logical_and(b == 0, j == 0))


def _call5(G, P, W2T, WoutT, bo2):
    return pl.pallas_call(
        _k5,
        grid=(B, NB),
        in_specs=[
            pl.BlockSpec((1, K, RB, CP), lambda b, j: (b, 0, j, 0)),
            pl.BlockSpec((1, RB, CP), lambda b, j: (b, j, 0)),
            pl.BlockSpec((CP, C), lambda b, j: (0, 0)),
            pl.BlockSpec((C, C), lambda b, j: (0, 0)),
            pl.BlockSpec((1, C), lambda b, j: (0, 0)),
        ],
        out_specs=[
            pl.BlockSpec((1, RB, C), lambda b, j: (b, j, 0)),
            pl.BlockSpec((8, C), lambda b, j: (0, 0)),
        ],
        out_shape=[
            jax.ShapeDtypeStruct((B, N, C), F32),
            jax.ShapeDtypeStruct((8, C), F32),
        ],
    )(G, P, W2T, WoutT, bo2)


def _k6(o_ref, s_ref, g_ref, be_ref, x_ref, f_ref):
    o = _bn_apply(s_ref, o_ref[0], g_ref, be_ref)
    f_ref[0] = jnp.maximum(o + x_ref[0], 0.0)


def _call6(O1, S2, g2, be2, x2):
    return pl.pallas_call(
        _k6,
        grid=(B, NB),
        in_specs=[
            pl.BlockSpec((1, RB, C), lambda b, j: (b, j, 0)),
            pl.BlockSpec((8, C), lambda b, j: (0, 0)),
            pl.BlockSpec((1, C), lambda b, j: (0, 0)),
            pl.BlockSpec((1, C), lambda b, j: (0, 0)),
            pl.BlockSpec((1, RB, C), lambda b, j: (b, j, 0)),
        ],
        out_specs=[pl.BlockSpec((1, RB, C), lambda b, j: (b, j, 0))],
        out_shape=[jax.ShapeDtypeStruct((B, N, C), F32)],
    )(O1, S2, g2, be2, x2)[0]


def kernel(x, W_in, b_in, g_in, be_in, W1, b1, W2, b2, W_out, b_out,
           g_out, be_out):
    Bx, Cc, Hh, Ww = x.shape
    x2 = x.reshape(Bx, Cc, Hh * Ww).transpose(0, 2, 1)
    W_inT = W_in.T
    pad = lambda w: jnp.pad(w, ((0, 0), (0, CP - Cc)))
    W1dT = pad((W1[:, :Cc] - W1[:, Cc:]).T)
    W1bT = pad(W1[:, Cc:].T)
    W2T = jnp.pad(W2.T, ((0, CP - Cc), (0, 0)))
    WoutT = W_out.T
    row = lambda v: v.reshape(1, Cc)

    H1, S1 = _call1(x2, W_inT, row(b_in))
    feats, P, Q = _call2(H1, S1, row(g_in), row(be_in), W1dT, W1bT,
                         jnp.pad(b1, (0, CP - Cc)).reshape(1, CP))
    idxT = _call3(feats)
    Gf = _sc_gather(Q.reshape(Bx * Hh * Ww, CP), idxT.reshape(NROWS))
    G = Gf.reshape(Bx, K, Hh * Ww, CP)
    bo_eff = (b_out + jnp.dot(b2, WoutT)).reshape(1, Cc)
    O1, S2 = _call5(G, P, W2T, WoutT, bo_eff)
    F = _call6(O1, S2, row(g_out), row(be_out), x2)
    return F.transpose(0, 2, 1).reshape(Bx, Cc, Hh, Ww)


# speedup vs baseline: 14.5532x; 1.6918x over previous
import functools

import jax
import jax.numpy as jnp
from jax import lax
from jax.experimental import pallas as pl
from jax.experimental.pallas import tpu as pltpu
from jax.experimental.pallas import tpu_sc as plsc

C = 192
CP = 256
HC = 128
K = 9
KE = K - 1
N = 1024
B = 8
RB = 1024
NB = N // RB
R3B = 1024
N3B = N // R3B
R5B = 1024
N5B = N // R5B
NTOT = B * N
F32 = jnp.float32


def _stats_accum(s_ref, h, is_first):
    s = jnp.sum(h, axis=0, keepdims=True)
    ss = jnp.sum(h * h, axis=0, keepdims=True)
    contrib = jnp.concatenate([s, ss, jnp.zeros((6, C), F32)], axis=0)

    @pl.when(is_first)
    def _():
        s_ref[...] = contrib

    @pl.when(jnp.logical_not(is_first))
    def _():
        s_ref[...] = s_ref[...] + contrib


def _bn_apply(s_ref, h, g_ref, be_ref):
    mean = s_ref[0:1, :] * (1.0 / NTOT)
    var = s_ref[1:2, :] * (1.0 / NTOT) - mean * mean
    inv = 1.0 / jnp.sqrt(var + 1e-5)
    return (h - mean) * inv * g_ref[...] + be_ref[...]


def _k1(x_ref, w_ref, b_ref, h_ref, s_ref):
    b = pl.program_id(0)
    j = pl.program_id(1)
    h = jnp.dot(x_ref[0], w_ref[...],
                preferred_element_type=F32) + b_ref[...]
    h_ref[0] = h
    _stats_accum(s_ref, h, jnp.logical_and(b == 0, j == 0))


def _call1(x2, W_inT, b_in2):
    return pl.pallas_call(
        _k1,
        grid=(B, NB),
        in_specs=[
            pl.BlockSpec((1, RB, C), lambda b, j: (b, j, 0)),
            pl.BlockSpec((C, C), lambda b, j: (0, 0)),
            pl.BlockSpec((1, C), lambda b, j: (0, 0)),
        ],
        out_specs=[
            pl.BlockSpec((1, RB, C), lambda b, j: (b, j, 0)),
            pl.BlockSpec((8, C), lambda b, j: (0, 0)),
        ],
        out_shape=[
            jax.ShapeDtypeStruct((B, N, C), F32),
            jax.ShapeDtypeStruct((8, C), F32),
        ],
    )(x2, W_inT, b_in2)


def _k2(h_ref, s_ref, g_ref, be_ref, wd_ref, wb_ref, b1_ref,
        f_ref, p_ref, q_ref):
    xh = _bn_apply(s_ref, h_ref[0], g_ref, be_ref)
    f = 0.5 * xh * (1.0 + lax.erf(xh * 0.7071067811865476))
    f_ref[0] = f
    zpad = jnp.zeros((C, CP - C), F32)
    wd = jnp.concatenate([wd_ref[...], zpad], axis=1)
    wb = jnp.concatenate([wb_ref[...], zpad], axis=1)
    b1p = jnp.concatenate([b1_ref[...], jnp.zeros((1, CP - C), F32)], axis=1)
    p_ref[0] = jnp.dot(f, wd, preferred_element_type=F32) + b1p
    q = jnp.dot(f, wb, preferred_element_type=F32)
    u = lax.bitcast_convert_type(q, jnp.uint32)
    top = (u + 0x7FFF + ((u >> 16) & 1)) & jnp.uint32(0xFFFF0000)
    lo, hi = top[:, :HC], top[:, HC:]
    q_ref[0] = lax.bitcast_convert_type(hi | (lo >> 16), jnp.int32)


def _call2(H1, S1, g2, be2, W1dT, W1bT, b12):
    return pl.pallas_call(
        _k2,
        grid=(B, NB),
        in_specs=[
            pl.BlockSpec((1, RB, C), lambda b, j: (b, j, 0)),
            pl.BlockSpec((8, C), lambda b, j: (0, 0)),
            pl.BlockSpec((1, C), lambda b, j: (0, 0)),
            pl.BlockSpec((1, C), lambda b, j: (0, 0)),
            pl.BlockSpec((C, C), lambda b, j: (0, 0)),
            pl.BlockSpec((C, C), lambda b, j: (0, 0)),
            pl.BlockSpec((1, C), lambda b, j: (0, 0)),
        ],
        out_specs=[
            pl.BlockSpec((1, RB, C), lambda b, j: (b, j, 0)),
            pl.BlockSpec((1, RB, CP), lambda b, j: (b, j, 0)),
            pl.BlockSpec((1, RB, HC), lambda b, j: (b, j, 0)),
        ],
        out_shape=[
            jax.ShapeDtypeStruct((B, N, C), F32),
            jax.ShapeDtypeStruct((B, N, CP), F32),
            jax.ShapeDtypeStruct((B, N, HC), jnp.int32),
        ],
    )(H1, S1, g2, be2, W1dT, W1bT, b12)


def _k3(b0, fall_ref, frow_ref, idx_ref):
    b = pl.program_id(0) + b0
    fa = fall_ref[0]
    fr = frow_ref[0]
    S = lax.dot_general(fr, fa, (((1,), (1,)), ((), ())),
                        preferred_element_type=F32)
    sqr = jnp.sum(fr * fr, axis=1, keepdims=True)
    sqa = jnp.sum(fa * fa, axis=1).reshape(1, N)
    d = jnp.maximum(sqr + sqa - 2.0 * S, 0.0)
    iota = lax.broadcasted_iota(jnp.int32, (R3B, N), 1)
    rowg = (lax.broadcasted_iota(jnp.int32, (R3B, N), 0)
            + pl.program_id(1) * R3B)
    d = jnp.where(iota == rowg, jnp.float32(3e38), d)
    base = b * N
    rows = []
    for _ in range(KE):
        m = jnp.min(d, axis=1, keepdims=True)
        c = d == m
        ik = jnp.min(jnp.where(c, iota, jnp.int32(2**30)), axis=1)
        rows.append(ik + base)
        d = jnp.where(c, jnp.float32(3e38), d)
    idx_ref[0] = jnp.stack(rows, axis=0)


HB = B // 2


def _call3(feats, b0):
    return pl.pallas_call(
        functools.partial(_k3, b0),
        grid=(HB, N3B),
        in_specs=[
            pl.BlockSpec((1, N, C), lambda b, j: (b + b0, 0, 0)),
            pl.BlockSpec((1, R3B, C), lambda b, j: (b + b0, j, 0)),
        ],
        out_specs=[pl.BlockSpec((1, KE, R3B), lambda b, j: (b, 0, j))],
        out_shape=[jax.ShapeDtypeStruct((HB, KE, N), jnp.int32)],
    )(feats, feats)[0]


NROWS = (B // 2) * KE * N
NW = 32
RPW = NROWS // NW
CH = 256
NCH = RPW // CH


def _sc_gather_body(q_hbm, idx_hbm, out_hbm, idx_v, rv0, rv1, s0, s1):
    wid = lax.axis_index("s") * 2 + lax.axis_index("c")
    base = wid * RPW
    pltpu.sync_copy(idx_hbm.at[wid // KE, wid % KE], idx_v)
    bufs, sems = (rv0, rv1), (s0, s1)
    pend = [pltpu.async_copy(q_hbm.at[idx_v.at[pl.ds(0, CH)]], rv0, s0)]
    for ch in range(NCH):
        if ch + 1 < NCH:
            pend.append(pltpu.async_copy(
                q_hbm.at[idx_v.at[pl.ds((ch + 1) * CH, CH)]],
                bufs[(ch + 1) % 2], sems[(ch + 1) % 2]))
        pend[ch].wait()
        pltpu.sync_copy(bufs[ch % 2], out_hbm.at[pl.ds(base + ch * CH, CH)])


@functools.lru_cache(maxsize=1)
def _sc_gather_fn():
    return pl.kernel(
        _sc_gather_body,
        out_type=jax.ShapeDtypeStruct((NROWS, HC), jnp.int32),
        mesh=plsc.VectorSubcoreMesh(core_axis_name="c", subcore_axis_name="s"),
        scratch_types=[
            pltpu.VMEM((RPW,), jnp.int32),
            pltpu.VMEM((CH, HC), jnp.int32),
            pltpu.VMEM((CH, HC), jnp.int32),
            pltpu.SemaphoreType.DMA,
            pltpu.SemaphoreType.DMA,
        ],
    )


def _sc_gather(q, idx):
    return _sc_gather_fn()(q, idx)


def _unpack(gp):
    u = lax.bitcast_convert_type(gp, jnp.uint32)
    lo = lax.bitcast_convert_type(u << 16, F32)
    hi = lax.bitcast_convert_type(u & jnp.uint32(0xFFFF0000), F32)
    return jnp.concatenate([lo, hi], axis=1)


def _k5(g_ref, q_ref, p_ref, w2_ref, wo_ref, bo_ref, o_ref, s_ref):
    b = pl.program_id(0)
    j = pl.program_id(1)
    p = p_ref[0]
    w2p = jnp.concatenate([w2_ref[...], jnp.zeros((CP - C, C), F32)], axis=0)
    acc = None
    for k in range(K):
        gp = q_ref[0] if k == 0 else g_ref[0, k - 1]
        e = _unpack(gp) + p
        e = jnp.where(e >= 0.0, e, 0.2 * e)
        y = jnp.dot(e, w2p, preferred_element_type=F32)
        acc = y if acc is None else jnp.maximum(acc, y)
    o = jnp.dot(acc, wo_ref[...],
                preferred_element_type=F32) + bo_ref[...]
    o_ref[0] = o
    _stats_accum(s_ref, o, jnp.logical_and(b == 0, j == 0))


def _call5(G, Q, P, W2T, WoutT, bo2, b0):
    return pl.pallas_call(
        _k5,
        grid=(HB, N5B),
        in_specs=[
            pl.BlockSpec((1, KE, R5B, HC), lambda b, j: (b, 0, j, 0)),
            pl.BlockSpec((1, R5B, HC), lambda b, j: (b + b0, j, 0)),
            pl.BlockSpec((1, R5B, CP), lambda b, j: (b + b0, j, 0)),
            pl.BlockSpec((C, C), lambda b, j: (0, 0)),
            pl.BlockSpec((C, C), lambda b, j: (0, 0)),
            pl.BlockSpec((1, C), lambda b, j: (0, 0)),
        ],
        out_specs=[
            pl.BlockSpec((1, R5B, C), lambda b, j: (b, j, 0)),
            pl.BlockSpec((8, C), lambda b, j: (0, 0)),
        ],
        out_shape=[
            jax.ShapeDtypeStruct((HB, N, C), F32),
            jax.ShapeDtypeStruct((8, C), F32),
        ],
    )(G, Q, P, W2T, WoutT, bo2)


def _k6(oa_ref, ob_ref, sa_ref, sb_ref, g_ref, be_ref, x_ref, f_ref):
    h = pl.program_id(0)
    o = jnp.where(h < HB, oa_ref[0], ob_ref[0])
    s = sa_ref[...] + sb_ref[...]
    mean = s[0:1, :] * (1.0 / NTOT)
    var = s[1:2, :] * (1.0 / NTOT) - mean * mean
    inv = 1.0 / jnp.sqrt(var + 1e-5)
    o = (o - mean) * inv * g_ref[...] + be_ref[...]
    f_ref[0] = jnp.maximum(o + x_ref[0], 0.0)


def _call6(O1a, O1b, S2a, S2b, g2, be2, x2):
    return pl.pallas_call(
        _k6,
        grid=(B, NB),
        in_specs=[
            pl.BlockSpec((1, RB, C), lambda b, j: (b % HB, j, 0)),
            pl.BlockSpec((1, RB, C), lambda b, j: (b % HB, j, 0)),
            pl.BlockSpec((8, C), lambda b, j: (0, 0)),
            pl.BlockSpec((8, C), lambda b, j: (0, 0)),
            pl.BlockSpec((1, C), lambda b, j: (0, 0)),
            pl.BlockSpec((1, C), lambda b, j: (0, 0)),
            pl.BlockSpec((1, RB, C), lambda b, j: (b, j, 0)),
        ],
        out_specs=[pl.BlockSpec((1, RB, C), lambda b, j: (b, j, 0))],
        out_shape=[jax.ShapeDtypeStruct((B, N, C), F32)],
    )(O1a, O1b, S2a, S2b, g2, be2, x2)[0]


def kernel(x, W_in, b_in, g_in, be_in, W1, b1, W2, b2, W_out, b_out,
           g_out, be_out):
    Bx, Cc, Hh, Ww = x.shape
    x2 = x.reshape(Bx, Cc, Hh * Ww).transpose(0, 2, 1)
    W_inT = W_in.T
    W1dT = (W1[:, :Cc] - W1[:, Cc:]).T
    W1bT = W1[:, Cc:].T
    W2T = W2.T
    WoutT = W_out.T
    row = lambda v: v.reshape(1, Cc)

    H1, S1 = _call1(x2, W_inT, row(b_in))
    feats, P, Q = _call2(H1, S1, row(g_in), row(be_in), W1dT, W1bT, row(b1))
    bo_eff = (b_out + jnp.dot(b2, WoutT)).reshape(1, Cc)
    Qf = Q.reshape(Bx * Hh * Ww, HC)
    idxA = _call3(feats, 0)
    GfA = _sc_gather(Qf, idxA)
    idxB = _call3(feats, HB)
    GfB = _sc_gather(Qf, idxB)
    GA = GfA.reshape(HB, KE, Hh * Ww, HC)
    GB = GfB.reshape(HB, KE, Hh * Ww, HC)
    O1a, S2a = _call5(GA, Q, P, W2T, WoutT, bo_eff, 0)
    O1b, S2b = _call5(GB, Q, P, W2T, WoutT, bo_eff, HB)
    F = _call6(O1a, O1b, S2a, S2b, row(g_out), row(be_out), x2)
    return F.transpose(0, 2, 1).reshape(Bx, Cc, Hh, Ww)
